# baseline (speedup 1.0000x reference)
import functools

import jax
import jax.numpy as jnp
import numpy as np
from jax import lax
from jax.experimental import pallas as pl
from jax.experimental.pallas import tpu as pltpu
from jax.experimental.pallas import tpu_sc as plsc

_NC = 2
_NS = 16
_NW = _NC * _NS
_CH = 128


def _sc_gather(nf, s_idx, r_idx):
    E = s_idx.shape[0]
    D = nf.shape[1]
    n_chunks = E // _CH
    nk = n_chunks // _NW
    rem = n_chunks % _NW
    mesh = plsc.VectorSubcoreMesh(core_axis_name="c", subcore_axis_name="s")

    @functools.partial(
        pl.kernel,
        out_type=(jax.ShapeDtypeStruct((E, D), jnp.float32),
                  jax.ShapeDtypeStruct((E, D), jnp.float32)),
        mesh=mesh,
        scratch_types=[
            pltpu.VMEM((_CH,), jnp.int32),
            pltpu.VMEM((_CH,), jnp.int32),
            pltpu.VMEM((_CH, D), jnp.float32),
            pltpu.VMEM((_CH, D), jnp.float32),
            pltpu.SemaphoreType.DMA,
            pltpu.SemaphoreType.DMA,
        ],
    )
    def k(nf_hbm, s_hbm, r_hbm, xs_hbm, xr_hbm, si_v, ri_v, sr_v, rr_v, sem_s, sem_r):
        wid = lax.axis_index("s") * _NC + lax.axis_index("c")

        def do_chunk(c):
            base = c * _CH
            pltpu.sync_copy(s_hbm.at[pl.ds(base, _CH)], si_v)
            pltpu.sync_copy(r_hbm.at[pl.ds(base, _CH)], ri_v)
            cp_s = pltpu.async_copy(nf_hbm.at[si_v], sr_v, sem_s)
            cp_r = pltpu.async_copy(nf_hbm.at[ri_v], rr_v, sem_r)
            cp_s.wait()
            cp_r.wait()
            pltpu.sync_copy(sr_v, xs_hbm.at[pl.ds(base, _CH)])
            pltpu.sync_copy(rr_v, xr_hbm.at[pl.ds(base, _CH)])

        @pl.loop(0, nk)
        def _(i):
            do_chunk(i * _NW + wid)

        if rem:
            @pl.when(wid < rem)
            def _():
                do_chunk(nk * _NW + wid)

    return k(nf, s_idx, r_idx)


_NSPLIT = 5120
_DUMP = _NSPLIT
_ACC_ROWS = 5248


def _sc_scatter_add(m, r_idx, n_nodes):
    E, C = m.shape
    n_chunks = E // _CH
    nk = n_chunks // _NS
    rem = n_chunks % _NS
    rows_per_sub = _ACC_ROWS // _NS
    zeros = jnp.zeros((_ACC_ROWS, C), jnp.float32)
    mesh = plsc.VectorSubcoreMesh(core_axis_name="c", subcore_axis_name="s")

    @functools.partial(
        pl.kernel,
        out_type=jax.ShapeDtypeStruct((_NC, _ACC_ROWS, C), jnp.float32),
        mesh=mesh,
        scratch_types=[
            pltpu.VMEM((_CH,), jnp.int32),
            pltpu.VMEM((_CH, C), jnp.float32),
            pltpu.VMEM_SHARED((_ACC_ROWS, C), jnp.float32),
            pltpu.SemaphoreType.DMA,
        ],
    )
    def k(m_hbm, r_hbm, z_hbm, out_hbm, idx_v, rows_v, acc_sh, sem):
        cid = lax.axis_index("c")
        sid = lax.axis_index("s")
        lo = cid * _NSPLIT

        pltpu.sync_copy(z_hbm.at[pl.ds(sid * rows_per_sub, rows_per_sub)],
                        acc_sh.at[pl.ds(sid * rows_per_sub, rows_per_sub)])
        plsc.subcore_barrier()

        def do_chunk(c):
            base = c * _CH
            pltpu.sync_copy(r_hbm.at[pl.ds(base, _CH)], idx_v)
            pltpu.sync_copy(m_hbm.at[pl.ds(base, _CH)], rows_v)
            for j in range(_CH // 16):
                raw = idx_v[pl.ds(j * 16, 16)]
                v = raw - lo
                oob = jnp.logical_or(v < 0, v >= _NSPLIT)
                idx_v[pl.ds(j * 16, 16)] = jnp.where(
                    oob, _DUMP + (raw & 127), v)
            pltpu.sync_copy(rows_v, acc_sh.at[idx_v], add=True)

        @pl.loop(0, nk)
        def _(i):
            do_chunk(i * _NS + sid)

        if rem:
            @pl.when(sid < rem)
            def _():
                do_chunk(nk * _NS + sid)

        plsc.subcore_barrier()
        pltpu.sync_copy(
            acc_sh.at[pl.ds(sid * rows_per_sub, rows_per_sub)],
            out_hbm.at[cid, pl.ds(sid * rows_per_sub, rows_per_sub)])

    return k(m, r_idx, zeros)


def _tc_edge_mlp(xs, xr, xe, w1s, w1r, w1e, b1, w2, b2):
    E, D = xs.shape
    DE = xe.shape[1]
    H = w1s.shape[1]
    C = w2.shape[1]
    BE = 2000

    def body(xs_ref, xr_ref, xe_ref, w1s_ref, w1r_ref, w1e_ref, b1_ref,
             w2_ref, b2_ref, o_ref):
        bf = jnp.bfloat16
        h = (jnp.dot(xs_ref[...].astype(bf), w1s_ref[...],
                     preferred_element_type=jnp.float32)
             + jnp.dot(xr_ref[...].astype(bf), w1r_ref[...],
                       preferred_element_type=jnp.float32)
             + jnp.dot(xe_ref[...].astype(bf), w1e_ref[...],
                       preferred_element_type=jnp.float32)
             + b1_ref[...])
        h = jax.nn.gelu(h)
        o_ref[...] = (jnp.dot(h.astype(bf), w2_ref[...],
                              preferred_element_type=jnp.float32)
                      + b2_ref[...])

    full = lambda s: pl.BlockSpec(s, lambda i: tuple(0 for _ in s))
    return pl.pallas_call(
        body,
        grid=(E // BE,),
        in_specs=[
            pl.BlockSpec((BE, D), lambda i: (i, 0)),
            pl.BlockSpec((BE, D), lambda i: (i, 0)),
            pl.BlockSpec((BE, DE), lambda i: (i, 0)),
            full((D, H)),
            full((D, H)),
            full((DE, H)),
            full((1, H)),
            full((H, C)),
            full((1, C)),
        ],
        out_specs=pl.BlockSpec((BE, C), lambda i: (i, 0)),
        out_shape=jax.ShapeDtypeStruct((E, C), jnp.float32),
        compiler_params=pltpu.CompilerParams(
            dimension_semantics=("arbitrary",)),
    )(xs, xr, xe, w1s, w1r, w1e, b1, w2, b2)


def _tc_node_mlp(m_cat, nf, w1m, w1x, b1, w2, b2, scale):
    N, D = nf.shape
    C = m_cat.shape[1]
    H = w1m.shape[1]
    DO = w2.shape[1]
    BN = 2000

    def body(m_ref, nf_ref, w1m_ref, w1x_ref, b1_ref, w2_ref, b2_ref, o_ref):
        m = m_ref[...] * scale
        h = (jnp.dot(m, w1m_ref[...], preferred_element_type=jnp.float32)
             + jnp.dot(nf_ref[...], w1x_ref[...], preferred_element_type=jnp.float32)
             + b1_ref[...])
        h = jax.nn.gelu(h)
        o_ref[...] = (jnp.dot(h, w2_ref[...], preferred_element_type=jnp.float32)
                      + b2_ref[...])

    full = lambda s: pl.BlockSpec(s, lambda i: tuple(0 for _ in s))
    return pl.pallas_call(
        body,
        grid=(N // BN,),
        in_specs=[
            pl.BlockSpec((BN, C), lambda i: (i, 0)),
            pl.BlockSpec((BN, D), lambda i: (i, 0)),
            full((C, H)),
            full((D, H)),
            full((1, H)),
            full((H, DO)),
            full((1, DO)),
        ],
        out_specs=pl.BlockSpec((BN, DO), lambda i: (i, 0)),
        out_shape=jax.ShapeDtypeStruct((N, DO), jnp.float32),
        compiler_params=pltpu.CompilerParams(
            dimension_semantics=("arbitrary",)),
    )(m_cat, nf, w1m, w1x, b1, w2, b2)


def kernel(node_features, edge_features, senders, receivers,
           W1_e, b1_e, W2_e, b2_e, W1_n, b1_n, W2_n, b2_n):
    nf = node_features[0]
    xe = edge_features[0]
    s_idx = senders[0].astype(jnp.int32)
    r_idx = receivers[0].astype(jnp.int32)
    N, D = nf.shape
    D_MSG = W2_e.shape[1]
    C = 128

    w1s = W1_e[:D]
    w1r = W1_e[D:2 * D]
    w1e = W1_e[2 * D:]
    b1e = b1_e[None, :]
    w2e = jnp.pad(W2_e, ((0, 0), (0, C - D_MSG)))
    b2e = jnp.pad(b2_e, (0, C - D_MSG))[None, :]

    w1m = jnp.pad(W1_n[:D_MSG], ((0, C - D_MSG), (0, 0)))
    w1x = W1_n[D_MSG:]
    b1n = b1_n[None, :]
    b2n = b2_n[None, :]

    xs, xr = _sc_gather(nf, s_idx, r_idx)
    bf = jnp.bfloat16
    m = _tc_edge_mlp(xs, xr, xe, w1s.astype(bf), w1r.astype(bf),
                     w1e.astype(bf), b1e, w2e.astype(bf), b2e)
    parts = _sc_scatter_add(m, r_idx, N)
    m_cat = jnp.concatenate([parts[0, :_NSPLIT], parts[1, :N - _NSPLIT]], axis=0)
    out = _tc_node_mlp(m_cat, nf, w1m, w1x, b1n, W2_n, b2n,
                       1.0 / np.sqrt(32.0))
    return out[None]


# speedup vs baseline: 1.2140x; 1.2140x over previous
import functools

import jax
import jax.numpy as jnp
import numpy as np
from jax import lax
from jax.experimental import pallas as pl
from jax.experimental.pallas import tpu as pltpu
from jax.experimental.pallas import tpu_sc as plsc

_NC = 2
_NS = 16
_NW = _NC * _NS
_CH = 128


def _sc_gather(nf, s_idx, r_idx):
    E = s_idx.shape[0]
    D = nf.shape[1]
    n_chunks = E // _CH
    nk = n_chunks // _NW
    rem = n_chunks % _NW
    mesh = plsc.VectorSubcoreMesh(core_axis_name="c", subcore_axis_name="s")

    @functools.partial(
        pl.kernel,
        out_type=(jax.ShapeDtypeStruct((E, D), jnp.float32),
                  jax.ShapeDtypeStruct((E, D), jnp.float32)),
        mesh=mesh,
        scratch_types=[
            pltpu.VMEM((_CH,), jnp.int32),
            pltpu.VMEM((_CH,), jnp.int32),
            pltpu.VMEM((_CH, D), jnp.float32),
            pltpu.VMEM((_CH, D), jnp.float32),
            pltpu.SemaphoreType.DMA,
            pltpu.SemaphoreType.DMA,
        ],
    )
    def k(nf_hbm, s_hbm, r_hbm, xs_hbm, xr_hbm, si_v, ri_v, sr_v, rr_v, sem_s, sem_r):
        wid = lax.axis_index("s") * _NC + lax.axis_index("c")

        def do_chunk(c):
            base = c * _CH
            pltpu.sync_copy(s_hbm.at[pl.ds(base, _CH)], si_v)
            pltpu.sync_copy(r_hbm.at[pl.ds(base, _CH)], ri_v)
            cp_s = pltpu.async_copy(nf_hbm.at[si_v], sr_v, sem_s)
            cp_r = pltpu.async_copy(nf_hbm.at[ri_v], rr_v, sem_r)
            cp_s.wait()
            cp_r.wait()
            pltpu.sync_copy(sr_v, xs_hbm.at[pl.ds(base, _CH)])
            pltpu.sync_copy(rr_v, xr_hbm.at[pl.ds(base, _CH)])

        @pl.loop(0, nk)
        def _(i):
            do_chunk(i * _NW + wid)

        if rem:
            @pl.when(wid < rem)
            def _():
                do_chunk(nk * _NW + wid)

    return k(nf, s_idx, r_idx)


_NSPLIT = 5120
_DUMP = _NSPLIT
_ACC_ROWS = 5248


def _sc_scatter_add(m, r_idx, n_nodes):
    E, C = m.shape
    n_chunks = E // _CH
    nk = n_chunks // _NS
    rem = n_chunks % _NS
    rows_per_sub = _ACC_ROWS // _NS
    zeros = jnp.zeros((_ACC_ROWS, C), jnp.float32)
    mesh = plsc.VectorSubcoreMesh(core_axis_name="c", subcore_axis_name="s")

    @functools.partial(
        pl.kernel,
        out_type=jax.ShapeDtypeStruct((_NC, _ACC_ROWS, C), jnp.float32),
        mesh=mesh,
        scratch_types=[
            pltpu.VMEM((_CH,), jnp.int32),
            pltpu.VMEM((_CH,), jnp.int32),
            pltpu.VMEM((_CH, C), jnp.float32),
            pltpu.VMEM((_CH, C), jnp.float32),
            pltpu.VMEM_SHARED((_ACC_ROWS, C), jnp.float32),
            pltpu.SemaphoreType.DMA,
            pltpu.SemaphoreType.DMA,
        ],
    )
    def k(m_hbm, r_hbm, z_hbm, out_hbm, idx_a, idx_b, rows_a, rows_b,
          acc_sh, sem_a, sem_b):
        cid = lax.axis_index("c")
        sid = lax.axis_index("s")
        lo = cid * _NSPLIT

        pltpu.sync_copy(z_hbm.at[pl.ds(sid * rows_per_sub, rows_per_sub)],
                        acc_sh.at[pl.ds(sid * rows_per_sub, rows_per_sub)])
        plsc.subcore_barrier()

        def start(i, idx_v, rows_v, sem):
            base = (i * _NS + sid) * _CH
            pltpu.async_copy(r_hbm.at[pl.ds(base, _CH)], idx_v, sem)
            pltpu.async_copy(m_hbm.at[pl.ds(base, _CH)], rows_v, sem)

        def wait(idx_v, rows_v, sem):
            pltpu.make_async_copy(r_hbm.at[pl.ds(0, _CH)], idx_v, sem).wait()
            pltpu.make_async_copy(m_hbm.at[pl.ds(0, _CH)], rows_v, sem).wait()

        def process(idx_v, rows_v):
            for j in range(_CH // 16):
                raw = idx_v[pl.ds(j * 16, 16)]
                v = raw - lo
                oob = jnp.logical_or(v < 0, v >= _NSPLIT)
                idx_v[pl.ds(j * 16, 16)] = jnp.where(
                    oob, _DUMP + (raw & 127), v)
            pltpu.sync_copy(rows_v, acc_sh.at[idx_v], add=True)

        start(0, idx_a, rows_a, sem_a)

        @pl.loop(0, nk // 2)
        def _(t):
            start(2 * t + 1, idx_b, rows_b, sem_b)
            wait(idx_a, rows_a, sem_a)
            process(idx_a, rows_a)

            @pl.when(2 * t + 2 < nk)
            def _():
                start(2 * t + 2, idx_a, rows_a, sem_a)

            wait(idx_b, rows_b, sem_b)
            process(idx_b, rows_b)

        if nk % 2:
            wait(idx_a, rows_a, sem_a)
            process(idx_a, rows_a)

        if rem:
            @pl.when(sid < rem)
            def _():
                base = (nk * _NS + sid) * _CH
                pltpu.sync_copy(r_hbm.at[pl.ds(base, _CH)], idx_a)
                pltpu.sync_copy(m_hbm.at[pl.ds(base, _CH)], rows_a)
                process(idx_a, rows_a)

        plsc.subcore_barrier()
        pltpu.sync_copy(
            acc_sh.at[pl.ds(sid * rows_per_sub, rows_per_sub)],
            out_hbm.at[cid, pl.ds(sid * rows_per_sub, rows_per_sub)])

    return k(m, r_idx, zeros)


def _tc_edge_mlp(xs, xr, xe, w1s, w1r, w1e, b1, w2, b2):
    E, D = xs.shape
    DE = xe.shape[1]
    H = w1s.shape[1]
    C = w2.shape[1]
    BE = 2000

    def body(xs_ref, xr_ref, xe_ref, w1s_ref, w1r_ref, w1e_ref, b1_ref,
             w2_ref, b2_ref, o_ref):
        h = (jnp.dot(xs_ref[...], w1s_ref[...], preferred_element_type=jnp.float32)
             + jnp.dot(xr_ref[...], w1r_ref[...], preferred_element_type=jnp.float32)
             + jnp.dot(xe_ref[...], w1e_ref[...], preferred_element_type=jnp.float32)
             + b1_ref[...])
        h = jax.nn.gelu(h)
        o_ref[...] = (jnp.dot(h, w2_ref[...], preferred_element_type=jnp.float32)
                      + b2_ref[...])

    full = lambda s: pl.BlockSpec(s, lambda i: tuple(0 for _ in s))
    return pl.pallas_call(
        body,
        grid=(E // BE,),
        in_specs=[
            pl.BlockSpec((BE, D), lambda i: (i, 0)),
            pl.BlockSpec((BE, D), lambda i: (i, 0)),
            pl.BlockSpec((BE, DE), lambda i: (i, 0)),
            full((D, H)),
            full((D, H)),
            full((DE, H)),
            full((1, H)),
            full((H, C)),
            full((1, C)),
        ],
        out_specs=pl.BlockSpec((BE, C), lambda i: (i, 0)),
        out_shape=jax.ShapeDtypeStruct((E, C), jnp.float32),
        compiler_params=pltpu.CompilerParams(
            dimension_semantics=("arbitrary",)),
    )(xs, xr, xe, w1s, w1r, w1e, b1, w2, b2)


def _tc_node_mlp(m_cat, nf, w1m, w1x, b1, w2, b2, scale):
    N, D = nf.shape
    C = m_cat.shape[1]
    H = w1m.shape[1]
    DO = w2.shape[1]
    BN = 2000

    def body(m_ref, nf_ref, w1m_ref, w1x_ref, b1_ref, w2_ref, b2_ref, o_ref):
        m = m_ref[...] * scale
        h = (jnp.dot(m, w1m_ref[...], preferred_element_type=jnp.float32)
             + jnp.dot(nf_ref[...], w1x_ref[...], preferred_element_type=jnp.float32)
             + b1_ref[...])
        h = jax.nn.gelu(h)
        o_ref[...] = (jnp.dot(h, w2_ref[...], preferred_element_type=jnp.float32)
                      + b2_ref[...])

    full = lambda s: pl.BlockSpec(s, lambda i: tuple(0 for _ in s))
    return pl.pallas_call(
        body,
        grid=(N // BN,),
        in_specs=[
            pl.BlockSpec((BN, C), lambda i: (i, 0)),
            pl.BlockSpec((BN, D), lambda i: (i, 0)),
            full((C, H)),
            full((D, H)),
            full((1, H)),
            full((H, DO)),
            full((1, DO)),
        ],
        out_specs=pl.BlockSpec((BN, DO), lambda i: (i, 0)),
        out_shape=jax.ShapeDtypeStruct((N, DO), jnp.float32),
        compiler_params=pltpu.CompilerParams(
            dimension_semantics=("arbitrary",)),
    )(m_cat, nf, w1m, w1x, b1, w2, b2)


def kernel(node_features, edge_features, senders, receivers,
           W1_e, b1_e, W2_e, b2_e, W1_n, b1_n, W2_n, b2_n):
    nf = node_features[0]
    xe = edge_features[0]
    s_idx = senders[0].astype(jnp.int32)
    r_idx = receivers[0].astype(jnp.int32)
    N, D = nf.shape
    D_MSG = W2_e.shape[1]
    C = 128

    w1s = W1_e[:D]
    w1r = W1_e[D:2 * D]
    w1e = W1_e[2 * D:]
    b1e = b1_e[None, :]
    w2e = jnp.pad(W2_e, ((0, 0), (0, C - D_MSG)))
    b2e = jnp.pad(b2_e, (0, C - D_MSG))[None, :]

    w1m = jnp.pad(W1_n[:D_MSG], ((0, C - D_MSG), (0, 0)))
    w1x = W1_n[D_MSG:]
    b1n = b1_n[None, :]
    b2n = b2_n[None, :]

    xs, xr = _sc_gather(nf, s_idx, r_idx)
    m = _tc_edge_mlp(xs, xr, xe, w1s, w1r, w1e, b1e, w2e, b2e)
    parts = _sc_scatter_add(m, r_idx, N)
    m_cat = jnp.concatenate([parts[0, :_NSPLIT], parts[1, :N - _NSPLIT]], axis=0)
    out = _tc_node_mlp(m_cat, nf, w1m, w1x, b1n, W2_n, b2n,
                       1.0 / np.sqrt(32.0))
    return out[None]
